# baseline (speedup 1.0000x reference)
import functools

import jax
import jax.numpy as jnp
from jax import lax
from jax.experimental import pallas as pl
from jax.experimental.pallas import tpu as pltpu
from jax.experimental.pallas import tpu_sc as plsc

B, S, D, K = 4, 4096, 2048, 512
SCHUNK = S // 8


def _tc_body(x_ref, w_ref, inds_ref, scale_ref, ycol_ref, yrow_ref):
    s = pl.program_id(1)
    w = w_ref[...]
    wn = w * lax.rsqrt(jnp.sum(w * w))
    xb = x_ref[0]
    yc = lax.dot_general(xb, wn, (((1,), (1,)), ((), ())),
                         preferred_element_type=jnp.float32)
    yr = lax.dot_general(wn, xb, (((1,), (1,)), ((), ())),
                         preferred_element_type=jnp.float32)
    ycol_ref[pl.ds(s * SCHUNK, SCHUNK), :] = yc
    yrow_ref[:, pl.ds(s * SCHUNK, SCHUNK)] = yr

    @pl.when(s == 7)
    def _topk():
        rslots = lax.broadcasted_iota(jnp.int32, (1, K), 1)
        inds_acc = jnp.zeros((1, K), jnp.int32)
        vals_acc = jnp.zeros((1, K), jnp.float32)
        for ic in range(S // SCHUNK):
            yi = ycol_ref[pl.ds(ic * SCHUNK, SCHUNK), :]
            icol = lax.broadcasted_iota(jnp.int32, (SCHUNK, 1), 0) + ic * SCHUNK
            cnt = jnp.zeros((SCHUNK, 1), jnp.int32)
            for jc in range(4):
                yj = yrow_ref[:, pl.ds(jc * 1024, 1024)]
                jj = lax.broadcasted_iota(jnp.int32, (1, 1024), 1) + jc * 1024
                gt_i = jnp.where(yj > yi, 1, 0)
                jlt_i = jnp.where(jj < icol, 1, 0)
                lex = jnp.where(yj == yi, jlt_i, gt_i)
                cnt = cnt + jnp.sum(lex, axis=1, keepdims=True)
            oh = cnt == rslots
            inds_acc = inds_acc + jnp.sum(
                jnp.where(oh, icol, 0), axis=0, keepdims=True)
            vals_acc = vals_acc + jnp.sum(
                jnp.where(oh, yi, 0.0), axis=0, keepdims=True)
        b1 = lax.broadcasted_iota(jnp.int32, (B, 1), 0)
        inds_ref[...] = inds_acc + b1 * S
        scale_ref[...] = jnp.broadcast_to(jnp.tanh(vals_acc), (16, K))[None]


_matvec_topk = pl.pallas_call(
    _tc_body,
    grid=(B, S // SCHUNK),
    in_specs=[
        pl.BlockSpec((1, SCHUNK, D), lambda b, s: (b, s, 0)),
        pl.BlockSpec((1, D), lambda b, s: (0, 0)),
    ],
    out_specs=[
        pl.BlockSpec((B, K), lambda b, s: (0, b)),
        pl.BlockSpec((1, 16, K), lambda b, s: (b, 0, 0)),
    ],
    out_shape=[
        jax.ShapeDtypeStruct((B, B * K), jnp.int32),
        jax.ShapeDtypeStruct((B, 16, K), jnp.float32),
    ],
    scratch_shapes=[
        pltpu.VMEM((S, 1), jnp.float32),
        pltpu.VMEM((1, S), jnp.float32),
    ],
    compiler_params=pltpu.CompilerParams(
        dimension_semantics=("arbitrary", "arbitrary")),
)


_NROWS = B * B * K
_RPW = _NROWS // 32
_CH = 16
_NCH = _RPW // _CH


_NB = 2


def _sc_body(x_hbm, idx_hbm, scl_hbm, out_hbm, idx_v, scl_v, buf, g0, g1,
             w0, w1):
    wid = lax.axis_index("s") * 2 + lax.axis_index("c")
    base = wid * _RPW
    pltpu.sync_copy(idx_hbm.at[pl.ds(base, _RPW)], idx_v)
    pltpu.sync_copy(scl_hbm.at[pl.ds(base, _RPW)], scl_v)
    gsem = [g0, g1]
    wsem = [w0, w1]
    ghandle = [None] * _NB
    whandle = [None] * _NB

    def start_gather(c):
        nb = c % _NB
        iv = idx_v[pl.ds(c * _CH, _CH)]
        ghandle[nb] = pltpu.async_copy(x_hbm.at[iv], buf.at[nb], gsem[nb])

    start_gather(0)
    for c in range(_NCH):
        nb = c % _NB
        if c + 1 < _NCH:
            if c + 1 >= 2:
                whandle[(c + 1) % _NB].wait()
            start_gather(c + 1)
        ghandle[nb].wait()
        svs = [scl_v[c * _CH + r] for r in range(_CH)]

        def mul_body(v, nb=nb, svs=svs):
            for r in range(_CH):
                buf[nb, r, pl.ds(v, 16)] = buf[nb, r, pl.ds(v, 16)] * svs[r]

        plsc.parallel_loop(0, D, step=16, unroll=2)(mul_body)
        whandle[nb] = pltpu.async_copy(
            buf.at[nb], out_hbm.at[pl.ds(base + c * _CH, _CH)], wsem[nb])
    for h in whandle:
        h.wait()


@functools.lru_cache(maxsize=1)
def _make_sc_gather():
    mesh = plsc.VectorSubcoreMesh(core_axis_name="c", subcore_axis_name="s")
    return functools.partial(
        pl.kernel,
        mesh=mesh,
        out_type=jax.ShapeDtypeStruct((_NROWS, D), jnp.float32),
        scratch_types=[
            pltpu.VMEM((_RPW,), jnp.int32),
            pltpu.VMEM((_RPW, 16), jnp.float32),
            pltpu.VMEM((_NB, _CH, D), jnp.float32),
            pltpu.SemaphoreType.DMA,
            pltpu.SemaphoreType.DMA,
            pltpu.SemaphoreType.DMA,
            pltpu.SemaphoreType.DMA,
        ],
    )(_sc_body)


def kernel(x, p, k):
    del k
    inds, scale = _matvec_topk(x, p.reshape(1, D))
    idx_flat = inds.reshape(_NROWS)
    scl = jnp.transpose(scale, (0, 2, 1)).reshape(B * K, 16)
    scl = jnp.tile(scl, (B, 1))
    x2 = x.reshape(B * S, D)
    out = _make_sc_gather()(x2, idx_flat, scl)
    return out.reshape(B, B, K, D)


# speedup vs baseline: 1.2013x; 1.2013x over previous
import functools

import jax
import jax.numpy as jnp
from jax import lax
from jax.experimental import pallas as pl
from jax.experimental.pallas import tpu as pltpu
from jax.experimental.pallas import tpu_sc as plsc

B, S, D, K = 4, 4096, 2048, 512
SCHUNK = S // 8


_NCHK = S // SCHUNK


def _tc_body(x_ref, w_ref, inds_ref, scale_ref, ycol_ref, ucol_ref):
    s = pl.program_id(1)
    w = w_ref[...]
    wn = w * lax.rsqrt(jnp.sum(w * w))
    xb = x_ref[0]
    yc = jnp.sum(xb * wn, axis=1, keepdims=True)
    yc = jnp.where(yc == 0.0, 0.0, yc)
    ycol_ref[pl.ds(s * SCHUNK, SCHUNK), :] = yc
    ub = lax.bitcast_convert_type(yc, jnp.int32)
    ucol_ref[pl.ds(s * SCHUNK, SCHUNK), :] = jnp.where(
        ub < 0, ub ^ jnp.int32(0x7FFFFFFF), ub)

    @pl.when(s == _NCHK - 1)
    def _topk():
        ii = lax.broadcasted_iota(jnp.int32, (K, K), 0)
        kk = lax.broadcasted_iota(jnp.int32, (K, K), 1)
        lt_tri = jnp.where(kk < ii, 1.0, 0.0)
        ident = jnp.where(kk == ii, 1.0, 0.0)
        rslots = lax.broadcasted_iota(jnp.int32, (1, K), 1)

        def uchunk(ic):
            return ucol_ref[pl.ds(ic * SCHUNK, SCHUNK), :]

        def cnt_ge(t):
            tot = jnp.int32(0)
            for ic in range(_NCHK):
                tot = tot + jnp.sum(jnp.where(uchunk(ic) >= t, 1, 0))
            return tot

        mn = jnp.int32(2147483647)
        mx = jnp.int32(-2147483648)
        for ic in range(_NCHK):
            mn = jnp.minimum(mn, jnp.min(uchunk(ic)))
            mx = jnp.maximum(mx, jnp.max(uchunk(ic)))

        def bs_body(_, c):
            lo, hi = c
            mid = (lo >> 1) + (hi >> 1) + (lo & hi & 1)
            big = cnt_ge(mid) >= K
            return (jnp.where(big, mid, lo), jnp.where(big, hi, mid))

        thr, _hi = lax.fori_loop(0, 32, bs_body, (mn, mx + 1))

        sgt, seq_ = [], []
        for ic in range(_NCHK):
            uc = uchunk(ic)
            sgt.append(jnp.sum(jnp.where(uc > thr, 1, 0)))
            seq_.append(jnp.sum(jnp.where(uc == thr, 1, 0)))
        n_gt = sgt[0]
        for v in sgt[1:]:
            n_gt = n_gt + v

        yrow_c = jnp.zeros((1, K), jnp.float32)
        irow_c = jnp.zeros((1, K), jnp.int32)
        off_gt = jnp.int32(0)
        off_eq = jnp.int32(0)
        for ic in range(_NCHK):
            uc = uchunk(ic)
            ycc = ycol_ref[pl.ds(ic * SCHUNK, SCHUNK), :]
            icol = (lax.broadcasted_iota(jnp.int32, (SCHUNK, 1), 0)
                    + ic * SCHUNK)
            mgt = uc > thr
            meq = uc == thr
            pgt = lax.dot_general(lt_tri, jnp.where(mgt, 1.0, 0.0),
                                  (((1,), (0,)), ((), ())),
                                  preferred_element_type=jnp.float32)
            peq = lax.dot_general(lt_tri, jnp.where(meq, 1.0, 0.0),
                                  (((1,), (0,)), ((), ())),
                                  preferred_element_type=jnp.float32)
            q = jnp.where(
                mgt, off_gt + pgt.astype(jnp.int32),
                jnp.where(meq, n_gt + off_eq + peq.astype(jnp.int32),
                          jnp.int32(K)))
            oh = q == rslots
            yrow_c = yrow_c + jnp.sum(jnp.where(oh, ycc, 0.0), axis=0,
                                      keepdims=True)
            irow_c = irow_c + jnp.sum(jnp.where(oh, icol, 0), axis=0,
                                      keepdims=True)
            off_gt = off_gt + sgt[ic]
            off_eq = off_eq + seq_[ic]

        ycol_c = lax.dot_general(ident, yrow_c, (((1,), (1,)), ((), ())),
                                 preferred_element_type=jnp.float32)
        icolf_c = lax.dot_general(ident, irow_c.astype(jnp.float32),
                                  (((1,), (1,)), ((), ())),
                                  preferred_element_type=jnp.float32)
        icol_c = icolf_c.astype(jnp.int32)

        gt_i = jnp.where(yrow_c > ycol_c, 1, 0)
        ilt_i = jnp.where(irow_c < icol_c, 1, 0)
        lex = jnp.where(yrow_c == ycol_c, ilt_i, gt_i)
        rc = jnp.sum(lex, axis=1, keepdims=True)
        oh2 = rc == rslots
        inds_row = jnp.sum(jnp.where(oh2, icol_c, 0), axis=0, keepdims=True)
        vals_row = jnp.sum(jnp.where(oh2, ycol_c, 0.0), axis=0, keepdims=True)
        b1 = lax.broadcasted_iota(jnp.int32, (B, 1), 0)
        inds_ref[...] = inds_row + b1 * S
        scale_ref[...] = jnp.broadcast_to(jnp.tanh(vals_row), (16, K))[None]


_matvec_topk = pl.pallas_call(
    _tc_body,
    grid=(B, S // SCHUNK),
    in_specs=[
        pl.BlockSpec((1, SCHUNK, D), lambda b, s: (b, s, 0)),
        pl.BlockSpec((1, D), lambda b, s: (0, 0)),
    ],
    out_specs=[
        pl.BlockSpec((B, K), lambda b, s: (0, b)),
        pl.BlockSpec((1, 16, K), lambda b, s: (b, 0, 0)),
    ],
    out_shape=[
        jax.ShapeDtypeStruct((B, B * K), jnp.int32),
        jax.ShapeDtypeStruct((B, 16, K), jnp.float32),
    ],
    scratch_shapes=[
        pltpu.VMEM((S, 1), jnp.float32),
        pltpu.VMEM((S, 1), jnp.int32),
    ],
    compiler_params=pltpu.CompilerParams(
        dimension_semantics=("arbitrary", "arbitrary")),
)


_NROWS = B * B * K
_RPW = _NROWS // 32
_CH = 16
_NCH = _RPW // _CH


_NB = 2


def _sc_body(x_hbm, idx_hbm, scl_hbm, out_hbm, idx_v, scl_v, buf, g0, g1,
             w0, w1):
    wid = lax.axis_index("s") * 2 + lax.axis_index("c")
    base = wid * _RPW
    pltpu.sync_copy(idx_hbm.at[pl.ds(base, _RPW)], idx_v)
    pltpu.sync_copy(scl_hbm.at[pl.ds(base, _RPW)], scl_v)
    gsem = [g0, g1]
    wsem = [w0, w1]
    ghandle = [None] * _NB
    whandle = [None] * _NB

    def start_gather(c):
        nb = c % _NB
        iv = idx_v[pl.ds(c * _CH, _CH)]
        ghandle[nb] = pltpu.async_copy(x_hbm.at[iv], buf.at[nb], gsem[nb])

    start_gather(0)
    for c in range(_NCH):
        nb = c % _NB
        if c + 1 < _NCH:
            if c + 1 >= 2:
                whandle[(c + 1) % _NB].wait()
            start_gather(c + 1)
        ghandle[nb].wait()
        svs = [scl_v[c * _CH + r] for r in range(_CH)]

        def mul_body(v, nb=nb, svs=svs):
            for r in range(_CH):
                buf[nb, r, pl.ds(v, 16)] = buf[nb, r, pl.ds(v, 16)] * svs[r]

        plsc.parallel_loop(0, D, step=16, unroll=2)(mul_body)
        whandle[nb] = pltpu.async_copy(
            buf.at[nb], out_hbm.at[pl.ds(base + c * _CH, _CH)], wsem[nb])
    for h in whandle:
        h.wait()


@functools.lru_cache(maxsize=1)
def _make_sc_gather():
    mesh = plsc.VectorSubcoreMesh(core_axis_name="c", subcore_axis_name="s")
    return functools.partial(
        pl.kernel,
        mesh=mesh,
        out_type=jax.ShapeDtypeStruct((_NROWS, D), jnp.float32),
        scratch_types=[
            pltpu.VMEM((_RPW,), jnp.int32),
            pltpu.VMEM((_RPW, 16), jnp.float32),
            pltpu.VMEM((_NB, _CH, D), jnp.float32),
            pltpu.SemaphoreType.DMA,
            pltpu.SemaphoreType.DMA,
            pltpu.SemaphoreType.DMA,
            pltpu.SemaphoreType.DMA,
        ],
    )(_sc_body)


def kernel(x, p, k):
    del k
    inds, scale = _matvec_topk(x, p.reshape(1, D))
    idx_flat = inds.reshape(_NROWS)
    scl = jnp.transpose(scale, (0, 2, 1)).reshape(B * K, 16)
    scl = jnp.tile(scl, (B, 1))
    x2 = x.reshape(B * S, D)
    out = _make_sc_gather()(x2, idx_flat, scl)
    return out.reshape(B, B, K, D)
